# Initial kernel scaffold; baseline (speedup 1.0000x reference)
import jax
import jax.numpy as jnp
from jax.experimental import pallas as pl


def kernel(inp, emb0, emb1, emb2, proj0, proj1, proj2):
    raise NotImplementedError("write your pallas kernel here")


# speedup vs baseline: 59.9649x; 59.9649x over previous
import jax
import jax.numpy as jnp
from jax.experimental import pallas as pl
from jax.experimental.pallas import tpu as pltpu
from jax.experimental.pallas import tpu_sc as plsc

_N_TOKEN = 1000000
_D_PROJ = 128
_EMB_SCALE = float(_D_PROJ) ** 0.5

_R = 4000
_B0_BLOCKS = 100000 // _R
_B1_BLOCKS = 300000 // _R
_B2_BLOCKS = 600000 // _R
_N_BLOCKS = _N_TOKEN // _R

_GATHER_WINDOW = 128


def _project_tables(emb0, emb1, emb2, p0t, p1t, p2t):
    def body(e0_ref, e1_ref, e2_ref, p0_ref, p1_ref, p2_ref, out_ref):
        i = pl.program_id(0)

        @pl.when(i < _B0_BLOCKS)
        def _():
            out_ref[...] = jnp.dot(
                e0_ref[...], p0_ref[...], preferred_element_type=jnp.float32
            )

        @pl.when(jnp.logical_and(i >= _B0_BLOCKS, i < _B0_BLOCKS + _B1_BLOCKS))
        def _():
            out_ref[...] = jnp.dot(
                e1_ref[...], p1_ref[...], preferred_element_type=jnp.float32
            )

        @pl.when(i >= _B0_BLOCKS + _B1_BLOCKS)
        def _():
            out_ref[...] = jnp.dot(
                e2_ref[...], p2_ref[...], preferred_element_type=jnp.float32
            )

    return pl.pallas_call(
        body,
        grid=(_N_BLOCKS,),
        in_specs=[
            pl.BlockSpec((_R, 128), lambda i: (jnp.minimum(i, _B0_BLOCKS - 1), 0)),
            pl.BlockSpec(
                (_R, 32),
                lambda i: (jnp.clip(i - _B0_BLOCKS, 0, _B1_BLOCKS - 1), 0),
            ),
            pl.BlockSpec(
                (_R, 8),
                lambda i: (
                    jnp.clip(i - _B0_BLOCKS - _B1_BLOCKS, 0, _B2_BLOCKS - 1),
                    0,
                ),
            ),
            pl.BlockSpec((128, 128), lambda i: (0, 0)),
            pl.BlockSpec((32, 128), lambda i: (0, 0)),
            pl.BlockSpec((8, 128), lambda i: (0, 0)),
        ],
        out_specs=pl.BlockSpec((_R, 128), lambda i: (i, 0)),
        out_shape=jax.ShapeDtypeStruct((_N_TOKEN, _D_PROJ), jnp.float32),
    )(emb0, emb1, emb2, p0t, p1t, p2t)


def _sc_gather(table, idx_flat):
    n = idx_flat.shape[1]
    mesh = plsc.VectorSubcoreMesh(core_axis_name="core", subcore_axis_name="subcore")

    @pl.kernel(
        out_type=jax.ShapeDtypeStruct((n, _D_PROJ), jnp.float32),
        mesh=mesh,
    )
    def k(tbl_hbm, i_hbm, o_hbm):
        def body(i_vmem, o_vmem):
            pltpu.sync_copy(tbl_hbm.at[i_vmem.at[0]], o_vmem)

        pltpu.emit_pipeline(
            body,
            grid=(n // _GATHER_WINDOW,),
            in_specs=[
                pl.BlockSpec((1, _GATHER_WINDOW), lambda i: (0, i)),
            ],
            out_specs=[
                pl.BlockSpec((_GATHER_WINDOW, _D_PROJ), lambda i: (i, 0)),
            ],
            core_axis_name=("core", "subcore"),
            dimension_semantics=(pltpu.PARALLEL,),
        )(i_hbm, o_hbm)

    return k(table, idx_flat)


def kernel(inp, emb0, emb1, emb2, proj0, proj1, proj2):
    scale = jnp.float32(_EMB_SCALE)
    p0t = proj0.T * scale
    p1t = proj1.T * scale
    p2t = proj2.T * scale
    table = _project_tables(emb0, emb1, emb2, p0t, p1t, p2t)
    idx_flat = inp.reshape(1, -1)
    out = _sc_gather(table, idx_flat)
    return out.reshape(inp.shape + (_D_PROJ,))
